# baseline (speedup 1.0000x reference)
import functools

import jax
import jax.numpy as jnp
from jax import lax
from jax.experimental import pallas as pl
from jax.experimental.pallas import tpu as pltpu
from jax.experimental.pallas import tpu_sc as plsc

_IDX_PAD = 128


def _sc_gather(table, idx):
    V, D = table.shape
    B = idx.shape[0]
    info = plsc.get_sparse_core_info()
    NC, NS = info.num_cores, info.num_subcores
    NW = NC * NS
    b_per_w = B // NW
    NC = 1
    NW = NC * NS
    b_per_w = B // NW
    mesh = plsc.VectorSubcoreMesh(core_axis_name="c", subcore_axis_name="s",
                                  num_cores=NC)

    @functools.partial(
        pl.kernel,
        mesh=mesh,
        out_type=jax.ShapeDtypeStruct((B, D), jnp.float32),
        scratch_types=[
            pltpu.VMEM((b_per_w,), jnp.int32),
            pltpu.VMEM((b_per_w, D), jnp.float32),
            pltpu.SemaphoreType.DMA,
        ],
    )
    def gather_kernel(table_hbm, idx_hbm, out_hbm, idx_v, rows_v, sem):
        wid = lax.axis_index("s") * NC + lax.axis_index("c")
        base = wid * b_per_w
        pltpu.sync_copy(idx_hbm.at[pl.ds(base, b_per_w)], idx_v)
        pltpu.async_copy(table_hbm.at[idx_v], rows_v, sem).wait()
        pltpu.sync_copy(rows_v, out_hbm.at[pl.ds(base, b_per_w)])

    return gather_kernel(table, idx)


def _tc_body(n_rel, pos_ref, len_ref, rel_ref, w_ref, b_ref, lam_ref,
             pabs_ref, out_ref):
    K = (n_rel - 1) // 2
    B = pos_ref.shape[0]
    p = pos_ref[...].reshape(1, B)
    L = len_ref[...].reshape(1, B)
    i = lax.broadcasted_iota(jnp.int32, (_IDX_PAD, 1), 0)

    c_lo = jnp.clip(p - (K - 1), 0, L)
    c_hi = jnp.maximum(L - (p + K), 0)
    mid = ((i >= 1) & (i <= 2 * K - 1) & (i >= K - p) & (i < L - p + K))
    cnt_t = jnp.where(i == 0, c_lo, jnp.where(i == 2 * K, c_hi,
                                              mid.astype(jnp.int32)))
    cnt_f = cnt_t.astype(jnp.float32)

    rel = jnp.concatenate(
        [rel_ref[...], jnp.zeros((_IDX_PAD - n_rel, rel_ref.shape[1]),
                                 jnp.float32)], axis=0)

    s = jnp.dot(rel, w_ref[...], preferred_element_type=jnp.float32)
    s = s + b_ref[0]
    s = jnp.where(i < n_rel, s, jnp.float32(-1e9))
    e = jnp.exp(s - jnp.max(s))

    dn = (((0,), (0,)), ((), ()))
    numer = lax.dot_general(cnt_f, e * rel, dn,
                            preferred_element_type=jnp.float32)
    z = lax.dot_general(cnt_f, e, dn,
                        preferred_element_type=jnp.float32)

    lam = lam_ref[0]
    out_ref[...] = lam * pabs_ref[...] + ((1.0 - lam) / z) * numer


def kernel(pos_ids, re_len, pe, rel_table, W_attn, b_attn, lam):
    B = pos_ids.shape[0]
    n_rel, H = rel_table.shape

    p_abs = _sc_gather(pe, pos_ids)

    grid = 2
    bb = B // grid
    smem = pl.BlockSpec(memory_space=pltpu.SMEM)
    fixed = lambda shape: pl.BlockSpec(shape, lambda g: (0,) * len(shape))
    return pl.pallas_call(
        functools.partial(_tc_body, n_rel),
        out_shape=jax.ShapeDtypeStruct((B, H), jnp.float32),
        grid=(grid,),
        in_specs=[
            pl.BlockSpec((bb,), lambda g: (g,)),
            pl.BlockSpec((bb,), lambda g: (g,)),
            fixed((n_rel, H)),
            fixed((H, 1)),
            smem,
            smem,
            pl.BlockSpec((bb, H), lambda g: (g, 0)),
        ],
        out_specs=pl.BlockSpec((bb, H), lambda g: (g, 0)),
    )(pos_ids, re_len, rel_table, W_attn, b_attn,
      jnp.asarray(lam, jnp.float32).reshape(1), p_abs)


# speedup vs baseline: 1.0132x; 1.0132x over previous
import functools

import jax
import jax.numpy as jnp
from jax import lax
from jax.experimental import pallas as pl
from jax.experimental.pallas import tpu as pltpu
from jax.experimental.pallas import tpu_sc as plsc

_IDX_PAD = 128


def _sc_gather(table, idx):
    V, D = table.shape
    B = idx.shape[0]
    info = plsc.get_sparse_core_info()
    NC, NS = info.num_cores, info.num_subcores
    NW = NC * NS
    b_per_w = B // NW
    NC = 1
    NW = NC * NS
    b_per_w = B // NW
    mesh = plsc.VectorSubcoreMesh(core_axis_name="c", subcore_axis_name="s",
                                  num_cores=NC)

    @functools.partial(
        pl.kernel,
        mesh=mesh,
        out_type=jax.ShapeDtypeStruct((B, D), jnp.float32),
        scratch_types=[
            pltpu.VMEM((b_per_w,), jnp.int32),
            pltpu.VMEM((b_per_w, D), jnp.float32),
            pltpu.SemaphoreType.DMA,
        ],
    )
    def gather_kernel(table_hbm, idx_hbm, out_hbm, idx_v, rows_v, sem):
        wid = lax.axis_index("s") * NC + lax.axis_index("c")
        base = wid * b_per_w
        pltpu.sync_copy(idx_hbm.at[pl.ds(base, b_per_w)], idx_v)
        pltpu.async_copy(table_hbm.at[idx_v], rows_v, sem).wait()
        pltpu.sync_copy(rows_v, out_hbm.at[pl.ds(base, b_per_w)])

    return gather_kernel(table, idx)


def _tc_body(n_rel, pos_ref, len_ref, rel_ref, w_ref, b_ref, lam_ref,
             pabs_ref, out_ref):
    K = (n_rel - 1) // 2
    B = pos_ref.shape[0]
    p = pos_ref[...].reshape(1, B)
    L = len_ref[...].reshape(1, B)
    i = lax.broadcasted_iota(jnp.int32, (_IDX_PAD, 1), 0)

    c_lo = jnp.clip(p - (K - 1), 0, L)
    c_hi = jnp.maximum(L - (p + K), 0)
    mid = ((i >= 1) & (i <= 2 * K - 1) & (i >= K - p) & (i < L - p + K))
    cnt_t = jnp.where(i == 0, c_lo, jnp.where(i == 2 * K, c_hi,
                                              mid.astype(jnp.int32)))
    cnt_f = cnt_t.astype(jnp.float32)

    rel = jnp.concatenate(
        [rel_ref[...], jnp.zeros((_IDX_PAD - n_rel, rel_ref.shape[1]),
                                 jnp.float32)], axis=0)

    s = jnp.dot(rel, w_ref[...], preferred_element_type=jnp.float32)
    s = s + b_ref[0]
    s = jnp.where(i < n_rel, s, jnp.float32(-1e9))
    e = jnp.exp(s - jnp.max(s))

    dn = (((0,), (0,)), ((), ()))
    numer = lax.dot_general(cnt_f, e * rel, dn,
                            preferred_element_type=jnp.float32)
    z = lax.dot_general(cnt_f, e, dn,
                        preferred_element_type=jnp.float32)

    lam = lam_ref[0]
    out_ref[...] = lam * pabs_ref[...] + ((1.0 - lam) / z) * numer


def kernel(pos_ids, re_len, pe, rel_table, W_attn, b_attn, lam):
    B = pos_ids.shape[0]
    n_rel, H = rel_table.shape

    p_abs = _sc_gather(pe, pos_ids)

    vmem = pl.BlockSpec(memory_space=pltpu.VMEM)
    smem = pl.BlockSpec(memory_space=pltpu.SMEM)
    return pl.pallas_call(
        functools.partial(_tc_body, n_rel),
        out_shape=jax.ShapeDtypeStruct((B, H), jnp.float32),
        in_specs=[vmem, vmem, vmem, vmem, smem, smem, vmem],
        out_specs=vmem,
    )(pos_ids, re_len, rel_table, W_attn, b_attn,
      jnp.asarray(lam, jnp.float32).reshape(1), p_abs)
